# baseline (speedup 1.0000x reference)
import functools

import jax
import jax.numpy as jnp
from jax import lax
from jax.experimental import pallas as pl
from jax.experimental.pallas import tpu as pltpu
from jax.experimental.pallas import tpu_sc as plsc

NC = 2
NS = 16
NW = NC * NS
CH = 80


def _layernorm(h, lw, lb):
    mu = jnp.mean(h, axis=-1, keepdims=True)
    d = h - mu
    var = jnp.mean(d * d, axis=-1, keepdims=True)
    return (d * lax.rsqrt(var + 1e-5)) * lw + lb


def _emb_body(x_ref, w1_ref, b1_ref, w2_ref, b2_ref, lw_ref, lb_ref, o_ref):
    h = jnp.dot(x_ref[...], w1_ref[...], preferred_element_type=jnp.float32)
    h = jax.nn.gelu(h + b1_ref[...])
    h = jnp.dot(h, w2_ref[...], preferred_element_type=jnp.float32) + b2_ref[...]
    o_ref[...] = _layernorm(h, lw_ref[...], lb_ref[...])


def _pack_bf16_halves(yf):
    b16 = lax.bitcast_convert_type(yf.astype(jnp.bfloat16), jnp.uint16)
    lo = b16[:, :64].astype(jnp.uint32)
    hi = b16[:, 64:].astype(jnp.uint32)
    return lax.bitcast_convert_type(lo | (hi << 16), jnp.int32)


def _unpack_pair_rows(w):
    wu = lax.bitcast_convert_type(w, jnp.uint32)
    lo = lax.bitcast_convert_type(wu << 16, jnp.float32)
    hi = lax.bitcast_convert_type(wu & jnp.uint32(0xFFFF0000), jnp.float32)
    even = jnp.concatenate([lo[:, :64], hi[:, :64]], axis=-1)
    odd = jnp.concatenate([lo[:, 64:], hi[:, 64:]], axis=-1)
    return jnp.concatenate([even, odd], axis=0)


def _ymm_body(x_ref, wd_ref, ws_ref, y1_ref, y2_ref):
    xb = x_ref[...]
    y1_ref[...] = _pack_bf16_halves(
        jnp.dot(xb, wd_ref[...], preferred_element_type=jnp.float32))
    y2_ref[...] = _pack_bf16_halves(
        jnp.dot(xb, ws_ref[...], preferred_element_type=jnp.float32))


def _edge_body(p1_ref, p2_ref, ea_ref, we_ref, b1_ref, w2_ref, b2_ref, lw_ref,
               lb_ref, o_ref):
    pre = _unpack_pair_rows(p1_ref[...]) + _unpack_pair_rows(p2_ref[...])
    h = jnp.dot(ea_ref[...], we_ref[...], preferred_element_type=jnp.float32)
    h = jax.nn.gelu(h + pre + b1_ref[...])
    h = jnp.dot(h, w2_ref[...], preferred_element_type=jnp.float32) + b2_ref[...]
    o_ref[...] = _layernorm(h, lw_ref[...], lb_ref[...])


def _edge1_body(p1_ref, p2_ref, ea16_ref, ew1_ref, eb1_ref, ew2_ref, eb2_ref,
                elw_ref, elb_ref, we_ref, b1_ref, w2_ref, b2_ref, lw_ref,
                lb_ref, o_ref):
    ea = jnp.dot(ea16_ref[...], ew1_ref[...], preferred_element_type=jnp.float32)
    ea = jax.nn.gelu(ea + eb1_ref[...])
    ea = jnp.dot(ea, ew2_ref[...], preferred_element_type=jnp.float32) + eb2_ref[...]
    ea = _layernorm(ea, elw_ref[...], elb_ref[...])
    pre = _unpack_pair_rows(p1_ref[...]) + _unpack_pair_rows(p2_ref[...])
    h = jnp.dot(ea, we_ref[...], preferred_element_type=jnp.float32)
    h = jax.nn.gelu(h + pre + b1_ref[...])
    h = jnp.dot(h, w2_ref[...], preferred_element_type=jnp.float32) + b2_ref[...]
    o_ref[...] = _layernorm(h, lw_ref[...], lb_ref[...])


def _node_body(x_ref, a0_ref, a1_ref, v1x_ref, v1a_ref, b1_ref, w2_ref, b2_ref,
               lw_ref, lb_ref, o_ref):
    xb = x_ref[...]
    agg = a0_ref[...] + a1_ref[...]
    h = (
        jnp.dot(xb, v1x_ref[...], preferred_element_type=jnp.float32)
        + jnp.dot(agg, v1a_ref[...], preferred_element_type=jnp.float32)
        + b1_ref[...]
    )
    h = jax.nn.gelu(h)
    h = jnp.dot(h, w2_ref[...], preferred_element_type=jnp.float32) + b2_ref[...]
    o_ref[...] = _layernorm(h, lw_ref[...], lb_ref[...]) + xb


def _node_y_body(x_ref, a0_ref, a1_ref, v1x_ref, v1a_ref, b1_ref, w2_ref,
                 b2_ref, lw_ref, lb_ref, wd_ref, ws_ref, o_ref, y1_ref, y2_ref):
    xb = x_ref[...]
    agg = a0_ref[...] + a1_ref[...]
    h = (
        jnp.dot(xb, v1x_ref[...], preferred_element_type=jnp.float32)
        + jnp.dot(agg, v1a_ref[...], preferred_element_type=jnp.float32)
        + b1_ref[...]
    )
    h = jax.nn.gelu(h)
    h = jnp.dot(h, w2_ref[...], preferred_element_type=jnp.float32) + b2_ref[...]
    xn = _layernorm(h, lw_ref[...], lb_ref[...]) + xb
    o_ref[...] = xn
    y1_ref[...] = _pack_bf16_halves(
        jnp.dot(xn, wd_ref[...], preferred_element_type=jnp.float32))
    y2_ref[...] = _pack_bf16_halves(
        jnp.dot(xn, ws_ref[...], preferred_element_type=jnp.float32))


def _row_spec(b, d):
    return pl.BlockSpec((b, d), lambda i: (i, 0))


def _full_spec(r, c):
    return pl.BlockSpec((r, c), lambda i: (0, 0))


def _tc_mlp(body, n_rows, block, row_ins, full_ins, n_out,
            out_dtype=jnp.float32, out_width=128):
    grid = n_rows // block
    in_specs = [_row_spec(block * a.shape[0] // n_rows, a.shape[-1])
                for a in row_ins]
    in_specs += [_full_spec(*a.shape) for a in full_ins]
    out_shape = [jax.ShapeDtypeStruct((n_rows, out_width), out_dtype)] * n_out
    out_specs = [_row_spec(block, out_width)] * n_out
    if n_out == 1:
        out_shape, out_specs = out_shape[0], out_specs[0]
    return pl.pallas_call(
        body,
        grid=(grid,),
        in_specs=in_specs,
        out_specs=out_specs,
        out_shape=out_shape,
    )(*row_ins, *full_ins)


def _sc_mesh():
    return plsc.VectorSubcoreMesh(core_axis_name="c", subcore_axis_name="s")


def _make_gather_add(n_edges):
    epw = n_edges // NW
    nchunk = epw // CH

    @functools.partial(
        pl.kernel,
        mesh=_sc_mesh(),
        out_type=[
            jax.ShapeDtypeStruct((n_edges // 2, 128), jnp.int32),
            jax.ShapeDtypeStruct((n_edges // 2, 128), jnp.int32),
        ],
        scratch_types=[
            pltpu.VMEM((nchunk, CH), jnp.int32),
            pltpu.VMEM((nchunk, CH), jnp.int32),
            pltpu.VMEM((2, CH, 64), jnp.int32),
            pltpu.VMEM((2, CH, 64), jnp.int32),
            pltpu.SemaphoreType.DMA((2,)),
            pltpu.SemaphoreType.DMA((2,)),
            pltpu.SemaphoreType.DMA((2,)),
            pltpu.SemaphoreType.DMA((2,)),
        ],
        name="sc_gather_pair",
        compiler_params=pltpu.CompilerParams(use_tc_tiling_on_sc=False),
    )
    def gather_pair(y1_hbm, y2_hbm, dst3_hbm, src3_hbm, o1_hbm, o2_hbm,
                    di_v, si_v, r1_v, r2_v, g1s, g2s, w1s, w2s):
        wid = lax.axis_index("s") * NC + lax.axis_index("c")
        base0 = wid * epw
        pltpu.sync_copy(dst3_hbm.at[wid], di_v)
        pltpu.sync_copy(src3_hbm.at[wid], si_v)
        pltpu.async_copy(y1_hbm.at[di_v.at[0]], r1_v.at[0], g1s.at[0])
        pltpu.async_copy(y2_hbm.at[si_v.at[0]], r2_v.at[0], g2s.at[0])

        def body(i, carry):
            b = lax.rem(i, 2)
            nb = 1 - b

            @pl.when(i + 1 < nchunk)
            def _issue_next():
                @pl.when(i >= 1)
                def _wb_done():
                    row_p = (base0 + (i - 1) * CH) // 2
                    for rv, ov, wsm in ((r1_v, o1_hbm, w1s), (r2_v, o2_hbm, w2s)):
                        pltpu.make_async_copy(
                            rv.at[nb, pl.ds(0, CH // 2)],
                            ov.at[pl.ds(row_p, CH // 2), pl.ds(0, 64)],
                            wsm.at[nb],
                        ).wait()
                        pltpu.make_async_copy(
                            rv.at[nb, pl.ds(CH // 2, CH // 2)],
                            ov.at[pl.ds(row_p, CH // 2), pl.ds(64, 64)],
                            wsm.at[nb],
                        ).wait()
                pltpu.async_copy(y1_hbm.at[di_v.at[i + 1]], r1_v.at[nb], g1s.at[nb])
                pltpu.async_copy(y2_hbm.at[si_v.at[i + 1]], r2_v.at[nb], g2s.at[nb])

            row_i = (base0 + i * CH) // 2
            pltpu.make_async_copy(
                y1_hbm.at[di_v.at[i]], r1_v.at[b], g1s.at[b]).wait()
            pltpu.async_copy(
                r1_v.at[b, pl.ds(0, CH // 2)],
                o1_hbm.at[pl.ds(row_i, CH // 2), pl.ds(0, 64)], w1s.at[b])
            pltpu.async_copy(
                r1_v.at[b, pl.ds(CH // 2, CH // 2)],
                o1_hbm.at[pl.ds(row_i, CH // 2), pl.ds(64, 64)], w1s.at[b])
            pltpu.make_async_copy(
                y2_hbm.at[si_v.at[i]], r2_v.at[b], g2s.at[b]).wait()
            pltpu.async_copy(
                r2_v.at[b, pl.ds(0, CH // 2)],
                o2_hbm.at[pl.ds(row_i, CH // 2), pl.ds(0, 64)], w2s.at[b])
            pltpu.async_copy(
                r2_v.at[b, pl.ds(CH // 2, CH // 2)],
                o2_hbm.at[pl.ds(row_i, CH // 2), pl.ds(64, 64)], w2s.at[b])
            return carry

        lax.fori_loop(0, nchunk, body, 0)
        for k in (nchunk - 2, nchunk - 1):
            row_k = (base0 + k * CH) // 2
            for rv, ov, wsm in ((r1_v, o1_hbm, w1s), (r2_v, o2_hbm, w2s)):
                pltpu.make_async_copy(
                    rv.at[k % 2, pl.ds(0, CH // 2)],
                    ov.at[pl.ds(row_k, CH // 2), pl.ds(0, 64)],
                    wsm.at[k % 2],
                ).wait()
                pltpu.make_async_copy(
                    rv.at[k % 2, pl.ds(CH // 2, CH // 2)],
                    ov.at[pl.ds(row_k, CH // 2), pl.ds(64, 64)],
                    wsm.at[k % 2],
                ).wait()

    return gather_pair


def _make_scatter_add(n_edges, n_nodes):
    epw = n_edges // NW
    nchunk = epw // CH
    rpw = (n_nodes // NS) // 8 * 8
    rem_base = rpw * NS
    rem = n_nodes - rem_base

    @functools.partial(
        pl.kernel,
        mesh=_sc_mesh(),
        out_type=jax.ShapeDtypeStruct((NC, n_nodes, 128), jnp.float32),
        scratch_types=[
            pltpu.VMEM((nchunk, CH), jnp.int32),
            pltpu.VMEM((2, CH, 128), jnp.float32),
            pltpu.VMEM_SHARED((n_nodes, 128), jnp.float32),
            pltpu.SemaphoreType.DMA((2,)),
        ],
        name="sc_scatter_add",
    )
    def scatter_add(en_hbm, dst3_hbm, zeros_hbm, out_hbm, di_v, rows_v, agg_sh,
                    lsem):
        cid = lax.axis_index("c")
        sid = lax.axis_index("s")
        wid = sid * NC + cid
        base0 = wid * epw
        pltpu.async_copy(en_hbm.at[pl.ds(base0, CH)], rows_v.at[0], lsem.at[0])
        pltpu.sync_copy(dst3_hbm.at[wid], di_v)
        pltpu.sync_copy(
            zeros_hbm.at[pl.ds(sid * rpw, rpw)], agg_sh.at[pl.ds(sid * rpw, rpw)]
        )
        if rem:
            @pl.when(sid == 0)
            def _zero_rem():
                pltpu.sync_copy(
                    zeros_hbm.at[pl.ds(rem_base, rem)],
                    agg_sh.at[pl.ds(rem_base, rem)],
                )
        plsc.subcore_barrier()

        def body(i, carry):
            b = lax.rem(i, 2)
            nb = 1 - b

            @pl.when(i + 1 < nchunk)
            def _issue_next():
                pltpu.async_copy(
                    en_hbm.at[pl.ds(base0 + (i + 1) * CH, CH)],
                    rows_v.at[nb], lsem.at[nb])

            pltpu.make_async_copy(
                en_hbm.at[pl.ds(base0 + i * CH, CH)],
                rows_v.at[b], lsem.at[b]).wait()
            pltpu.sync_copy(rows_v.at[b], agg_sh.at[di_v.at[i]], add=True)
            return carry

        lax.fori_loop(0, nchunk, body, 0)
        plsc.subcore_barrier()
        pltpu.sync_copy(
            agg_sh.at[pl.ds(sid * rpw, rpw)],
            out_hbm.at[cid].at[pl.ds(sid * rpw, rpw)],
        )
        if rem:
            @pl.when(sid == 0)
            def _out_rem():
                pltpu.sync_copy(
                    agg_sh.at[pl.ds(rem_base, rem)],
                    out_hbm.at[cid].at[pl.ds(rem_base, rem)],
                )

    return scatter_add


def kernel(x, edge_index, edge_attr_base, trainable, params, batch_size):
    n, c = x.shape
    e = edge_index.shape[1]
    src = edge_index[0]
    dst = edge_index[1]

    pe = params["emb_edges"]
    ea16 = jnp.concatenate(
        [edge_attr_base, trainable, jnp.zeros((e, 4), jnp.float32)], axis=-1
    )
    w1p = jnp.concatenate([pe["w1"], jnp.zeros((4, c), jnp.float32)], axis=0)

    gather_add = _make_gather_add(e)
    scatter_add = _make_scatter_add(e, n)
    zeros_nc = jnp.zeros((n, c), jnp.float32)
    epw = e // NW
    nchunk = epw // CH
    dst3 = dst.reshape(NW, nchunk, CH)
    src3 = src.reshape(NW, nchunk, CH)

    eb = 2000

    def _kperm(a):
        return (a.reshape(-1, eb // 2, 2, *a.shape[1:])
                .swapaxes(1, 2)
                .reshape(a.shape))

    ea16 = _kperm(ea16)
    dstk3 = _kperm(dst).reshape(NW, nchunk, CH)

    def _gperm(a):
        return (a.reshape(-1, CH // 2, 2).swapaxes(1, 2)
                .reshape(NW, nchunk, CH))

    dst3g = _gperm(dst)
    src3g = _gperm(src)

    blocks = params["blocks"]
    em0 = blocks[0]["edge_mlp"]
    y1, y2 = _tc_mlp(
        _ymm_body, n, 2000,
        [x],
        [em0["w1"][:c], em0["w1"][c:2 * c]],
        2, out_dtype=jnp.int32, out_width=64,
    )
    en = None
    for li, bp in enumerate(blocks):
        em, nm = bp["edge_mlp"], bp["node_mlp"]
        p1, p2 = gather_add(y1, y2, dst3g, src3g)
        edge_fulls = [em["w1"][2 * c:], em["b1"][None, :], em["w2"],
                      em["b2"][None, :], em["ln_w"][None, :],
                      em["ln_b"][None, :]]
        if li == 0:
            en = _tc_mlp(
                _edge1_body, e, 2000,
                [p1, p2, ea16],
                [w1p, pe["b1"][None, :], pe["w2"], pe["b2"][None, :],
                 pe["ln_w"][None, :], pe["ln_b"][None, :]] + edge_fulls,
                1,
            )
        else:
            en = _tc_mlp(_edge_body, e, 2000, [p1, p2, en], edge_fulls, 1)
        agg2 = scatter_add(en, dstk3, zeros_nc)
        node_fulls = [nm["w1"][:c], nm["w1"][c:], nm["b1"][None, :], nm["w2"],
                      nm["b2"][None, :], nm["ln_w"][None, :],
                      nm["ln_b"][None, :]]
        if li + 1 < len(blocks):
            emn = blocks[li + 1]["edge_mlp"]
            grid = n // 2000
            x, y1, y2 = pl.pallas_call(
                _node_y_body,
                grid=(grid,),
                in_specs=[_row_spec(2000, 128)] * 3
                + [_full_spec(*a.shape)
                   for a in node_fulls + [emn["w1"][:c], emn["w1"][c:2 * c]]],
                out_specs=[_row_spec(2000, 128), _row_spec(2000, 64),
                           _row_spec(2000, 64)],
                out_shape=[
                    jax.ShapeDtypeStruct((n, 128), jnp.float32),
                    jax.ShapeDtypeStruct((n, 64), jnp.int32),
                    jax.ShapeDtypeStruct((n, 64), jnp.int32),
                ],
            )(x, agg2[0], agg2[1], *node_fulls, emn["w1"][:c],
              emn["w1"][c:2 * c])
        else:
            x = _tc_mlp(_node_body, n, 2000, [x, agg2[0], agg2[1]],
                        node_fulls, 1)
    return x


# speedup vs baseline: 1.1669x; 1.1669x over previous
import functools

import jax
import jax.numpy as jnp
from jax import lax
from jax.experimental import pallas as pl
from jax.experimental.pallas import tpu as pltpu
from jax.experimental.pallas import tpu_sc as plsc

NC = 2
NS = 16
NW = NC * NS
CH = 80


def _layernorm(h, lw, lb):
    mu = jnp.mean(h, axis=-1, keepdims=True)
    d = h - mu
    var = jnp.mean(d * d, axis=-1, keepdims=True)
    return (d * lax.rsqrt(var + 1e-5)) * lw + lb


def _emb_body(x_ref, w1_ref, b1_ref, w2_ref, b2_ref, lw_ref, lb_ref, o_ref):
    h = jnp.dot(x_ref[...], w1_ref[...], preferred_element_type=jnp.float32)
    h = jax.nn.gelu(h + b1_ref[...])
    h = jnp.dot(h, w2_ref[...], preferred_element_type=jnp.float32) + b2_ref[...]
    o_ref[...] = _layernorm(h, lw_ref[...], lb_ref[...])


def _pack_bf16_halves(yf):
    b16 = lax.bitcast_convert_type(yf.astype(jnp.bfloat16), jnp.uint16)
    lo = b16[:, :64].astype(jnp.uint32)
    hi = b16[:, 64:].astype(jnp.uint32)
    return lax.bitcast_convert_type(lo | (hi << 16), jnp.int32)


def _unpack_pair_rows(w):
    wu = lax.bitcast_convert_type(w, jnp.uint32)
    lo = lax.bitcast_convert_type(wu << 16, jnp.float32)
    hi = lax.bitcast_convert_type(wu & jnp.uint32(0xFFFF0000), jnp.float32)
    even = jnp.concatenate([lo[:, :64], hi[:, :64]], axis=-1)
    odd = jnp.concatenate([lo[:, 64:], hi[:, 64:]], axis=-1)
    return jnp.concatenate([even, odd], axis=0)


def _ymm_body(x_ref, wd_ref, ws_ref, y1_ref, y2_ref):
    xb = x_ref[...]
    y1_ref[...] = _pack_bf16_halves(
        jnp.dot(xb, wd_ref[...], preferred_element_type=jnp.float32))
    y2_ref[...] = _pack_bf16_halves(
        jnp.dot(xb, ws_ref[...], preferred_element_type=jnp.float32))


def _edge_body(p1_ref, p2_ref, ea_ref, we_ref, b1_ref, w2_ref, b2_ref, lw_ref,
               lb_ref, o_ref):
    pre = _unpack_pair_rows(p1_ref[...]) + _unpack_pair_rows(p2_ref[...])
    h = jnp.dot(ea_ref[...], we_ref[...], preferred_element_type=jnp.float32)
    h = jax.nn.gelu(h + pre + b1_ref[...])
    h = jnp.dot(h, w2_ref[...], preferred_element_type=jnp.float32) + b2_ref[...]
    o_ref[...] = _layernorm(h, lw_ref[...], lb_ref[...])


def _edge1_body(p1_ref, p2_ref, ea16_ref, ew1_ref, eb1_ref, ew2_ref, eb2_ref,
                elw_ref, elb_ref, we_ref, b1_ref, w2_ref, b2_ref, lw_ref,
                lb_ref, o_ref):
    ea = jnp.dot(ea16_ref[...], ew1_ref[...], preferred_element_type=jnp.float32)
    ea = jax.nn.gelu(ea + eb1_ref[...])
    ea = jnp.dot(ea, ew2_ref[...], preferred_element_type=jnp.float32) + eb2_ref[...]
    ea = _layernorm(ea, elw_ref[...], elb_ref[...])
    pre = _unpack_pair_rows(p1_ref[...]) + _unpack_pair_rows(p2_ref[...])
    h = jnp.dot(ea, we_ref[...], preferred_element_type=jnp.float32)
    h = jax.nn.gelu(h + pre + b1_ref[...])
    h = jnp.dot(h, w2_ref[...], preferred_element_type=jnp.float32) + b2_ref[...]
    o_ref[...] = _layernorm(h, lw_ref[...], lb_ref[...])


def _node_body(x_ref, a0_ref, a1_ref, v1x_ref, v1a_ref, b1_ref, w2_ref, b2_ref,
               lw_ref, lb_ref, o_ref):
    xb = x_ref[...]
    agg = a0_ref[...] + a1_ref[...]
    h = (
        jnp.dot(xb, v1x_ref[...], preferred_element_type=jnp.float32)
        + jnp.dot(agg, v1a_ref[...], preferred_element_type=jnp.float32)
        + b1_ref[...]
    )
    h = jax.nn.gelu(h)
    h = jnp.dot(h, w2_ref[...], preferred_element_type=jnp.float32) + b2_ref[...]
    o_ref[...] = _layernorm(h, lw_ref[...], lb_ref[...]) + xb


def _node_y_body(x_ref, a0_ref, a1_ref, v1x_ref, v1a_ref, b1_ref, w2_ref,
                 b2_ref, lw_ref, lb_ref, wd_ref, ws_ref, o_ref, y1_ref, y2_ref):
    xb = x_ref[...]
    agg = a0_ref[...] + a1_ref[...]
    h = (
        jnp.dot(xb, v1x_ref[...], preferred_element_type=jnp.float32)
        + jnp.dot(agg, v1a_ref[...], preferred_element_type=jnp.float32)
        + b1_ref[...]
    )
    h = jax.nn.gelu(h)
    h = jnp.dot(h, w2_ref[...], preferred_element_type=jnp.float32) + b2_ref[...]
    xn = _layernorm(h, lw_ref[...], lb_ref[...]) + xb
    o_ref[...] = xn
    y1_ref[...] = _pack_bf16_halves(
        jnp.dot(xn, wd_ref[...], preferred_element_type=jnp.float32))
    y2_ref[...] = _pack_bf16_halves(
        jnp.dot(xn, ws_ref[...], preferred_element_type=jnp.float32))


def _row_spec(b, d):
    return pl.BlockSpec((b, d), lambda i: (i, 0))


def _full_spec(r, c):
    return pl.BlockSpec((r, c), lambda i: (0, 0))


def _tc_mlp(body, n_rows, block, row_ins, full_ins, n_out,
            out_dtype=jnp.float32, out_width=128):
    grid = n_rows // block
    in_specs = [_row_spec(block * a.shape[0] // n_rows, a.shape[-1])
                for a in row_ins]
    in_specs += [_full_spec(*a.shape) for a in full_ins]
    out_shape = [jax.ShapeDtypeStruct((n_rows, out_width), out_dtype)] * n_out
    out_specs = [_row_spec(block, out_width)] * n_out
    if n_out == 1:
        out_shape, out_specs = out_shape[0], out_specs[0]
    return pl.pallas_call(
        body,
        grid=(grid,),
        in_specs=in_specs,
        out_specs=out_specs,
        out_shape=out_shape,
    )(*row_ins, *full_ins)


def _sc_mesh():
    return plsc.VectorSubcoreMesh(core_axis_name="c", subcore_axis_name="s")


def _make_gather_add(n_edges):
    epw = n_edges // NW
    nchunk = epw // CH

    @functools.partial(
        pl.kernel,
        mesh=_sc_mesh(),
        out_type=[
            jax.ShapeDtypeStruct((n_edges, 64), jnp.int32),
            jax.ShapeDtypeStruct((n_edges, 64), jnp.int32),
        ],
        scratch_types=[
            pltpu.VMEM((nchunk, CH), jnp.int32),
            pltpu.VMEM((nchunk, CH), jnp.int32),
            pltpu.VMEM((2, CH, 64), jnp.int32),
            pltpu.VMEM((2, CH, 64), jnp.int32),
            pltpu.SemaphoreType.DMA((2,)),
            pltpu.SemaphoreType.DMA((2,)),
            pltpu.SemaphoreType.DMA((2,)),
            pltpu.SemaphoreType.DMA((2,)),
        ],
        name="sc_gather_pair",
        compiler_params=pltpu.CompilerParams(use_tc_tiling_on_sc=False),
    )
    def gather_pair(y1_hbm, y2_hbm, dst3_hbm, src3_hbm, o1_hbm, o2_hbm,
                    di_v, si_v, r1_v, r2_v, g1s, g2s, w1s, w2s):
        wid = lax.axis_index("s") * NC + lax.axis_index("c")
        base0 = wid * epw
        pltpu.sync_copy(dst3_hbm.at[wid], di_v)
        pltpu.sync_copy(src3_hbm.at[wid], si_v)
        pltpu.async_copy(y1_hbm.at[di_v.at[0]], r1_v.at[0], g1s.at[0])
        pltpu.async_copy(y2_hbm.at[si_v.at[0]], r2_v.at[0], g2s.at[0])

        def body(i, carry):
            b = lax.rem(i, 2)
            nb = 1 - b

            @pl.when(i + 1 < nchunk)
            def _issue_next():
                @pl.when(i >= 1)
                def _wb_done():
                    pltpu.make_async_copy(
                        r1_v.at[nb],
                        o1_hbm.at[pl.ds(base0 + (i - 1) * CH, CH)],
                        w1s.at[nb],
                    ).wait()
                    pltpu.make_async_copy(
                        r2_v.at[nb],
                        o2_hbm.at[pl.ds(base0 + (i - 1) * CH, CH)],
                        w2s.at[nb],
                    ).wait()
                pltpu.async_copy(y1_hbm.at[di_v.at[i + 1]], r1_v.at[nb], g1s.at[nb])
                pltpu.async_copy(y2_hbm.at[si_v.at[i + 1]], r2_v.at[nb], g2s.at[nb])

            pltpu.make_async_copy(
                y1_hbm.at[di_v.at[i]], r1_v.at[b], g1s.at[b]).wait()
            pltpu.async_copy(
                r1_v.at[b], o1_hbm.at[pl.ds(base0 + i * CH, CH)], w1s.at[b])
            pltpu.make_async_copy(
                y2_hbm.at[si_v.at[i]], r2_v.at[b], g2s.at[b]).wait()
            pltpu.async_copy(
                r2_v.at[b], o2_hbm.at[pl.ds(base0 + i * CH, CH)], w2s.at[b])
            return carry

        lax.fori_loop(0, nchunk, body, 0)
        for k in (nchunk - 2, nchunk - 1):
            pltpu.make_async_copy(
                r1_v.at[k % 2],
                o1_hbm.at[pl.ds(base0 + k * CH, CH)],
                w1s.at[k % 2],
            ).wait()
            pltpu.make_async_copy(
                r2_v.at[k % 2],
                o2_hbm.at[pl.ds(base0 + k * CH, CH)],
                w2s.at[k % 2],
            ).wait()

    return gather_pair


def _make_scatter_add(n_edges, n_nodes):
    epw = n_edges // NW
    nchunk = epw // CH
    rpw = (n_nodes // NS) // 8 * 8
    rem_base = rpw * NS
    rem = n_nodes - rem_base

    @functools.partial(
        pl.kernel,
        mesh=_sc_mesh(),
        out_type=jax.ShapeDtypeStruct((NC, n_nodes, 128), jnp.float32),
        scratch_types=[
            pltpu.VMEM((nchunk, CH), jnp.int32),
            pltpu.VMEM((2, CH, 128), jnp.float32),
            pltpu.VMEM_SHARED((n_nodes, 128), jnp.float32),
            pltpu.SemaphoreType.DMA((2,)),
        ],
        name="sc_scatter_add",
    )
    def scatter_add(en_hbm, dst3_hbm, zeros_hbm, out_hbm, di_v, rows_v, agg_sh,
                    lsem):
        cid = lax.axis_index("c")
        sid = lax.axis_index("s")
        wid = sid * NC + cid
        base0 = wid * epw
        pltpu.async_copy(en_hbm.at[pl.ds(base0, CH)], rows_v.at[0], lsem.at[0])
        pltpu.sync_copy(dst3_hbm.at[wid], di_v)
        pltpu.sync_copy(
            zeros_hbm.at[pl.ds(sid * rpw, rpw)], agg_sh.at[pl.ds(sid * rpw, rpw)]
        )
        if rem:
            @pl.when(sid == 0)
            def _zero_rem():
                pltpu.sync_copy(
                    zeros_hbm.at[pl.ds(rem_base, rem)],
                    agg_sh.at[pl.ds(rem_base, rem)],
                )
        plsc.subcore_barrier()

        def body(i, carry):
            b = lax.rem(i, 2)
            nb = 1 - b

            @pl.when(i + 1 < nchunk)
            def _issue_next():
                pltpu.async_copy(
                    en_hbm.at[pl.ds(base0 + (i + 1) * CH, CH)],
                    rows_v.at[nb], lsem.at[nb])

            pltpu.make_async_copy(
                en_hbm.at[pl.ds(base0 + i * CH, CH)],
                rows_v.at[b], lsem.at[b]).wait()
            pltpu.sync_copy(rows_v.at[b], agg_sh.at[di_v.at[i]], add=True)
            return carry

        lax.fori_loop(0, nchunk, body, 0)
        plsc.subcore_barrier()
        pltpu.sync_copy(
            agg_sh.at[pl.ds(sid * rpw, rpw)],
            out_hbm.at[cid].at[pl.ds(sid * rpw, rpw)],
        )
        if rem:
            @pl.when(sid == 0)
            def _out_rem():
                pltpu.sync_copy(
                    agg_sh.at[pl.ds(rem_base, rem)],
                    out_hbm.at[cid].at[pl.ds(rem_base, rem)],
                )

    return scatter_add


def kernel(x, edge_index, edge_attr_base, trainable, params, batch_size):
    n, c = x.shape
    e = edge_index.shape[1]
    src = edge_index[0]
    dst = edge_index[1]

    pe = params["emb_edges"]
    ea16 = jnp.concatenate(
        [edge_attr_base, trainable, jnp.zeros((e, 4), jnp.float32)], axis=-1
    )
    w1p = jnp.concatenate([pe["w1"], jnp.zeros((4, c), jnp.float32)], axis=0)

    gather_add = _make_gather_add(e)
    scatter_add = _make_scatter_add(e, n)
    zeros_nc = jnp.zeros((n, c), jnp.float32)
    epw = e // NW
    nchunk = epw // CH
    dst3 = dst.reshape(NW, nchunk, CH)
    src3 = src.reshape(NW, nchunk, CH)

    eb = 2000

    def _kperm(a):
        return (a.reshape(-1, eb // 2, 2, *a.shape[1:])
                .swapaxes(1, 2)
                .reshape(a.shape))

    ea16 = _kperm(ea16)
    dstk3 = _kperm(dst).reshape(NW, nchunk, CH)

    blocks = params["blocks"]
    em0 = blocks[0]["edge_mlp"]
    y1, y2 = _tc_mlp(
        _ymm_body, n, 2000,
        [x],
        [em0["w1"][:c], em0["w1"][c:2 * c]],
        2, out_dtype=jnp.int32, out_width=64,
    )
    en = None
    for li, bp in enumerate(blocks):
        em, nm = bp["edge_mlp"], bp["node_mlp"]
        p1, p2 = gather_add(y1, y2, dst3, src3)
        p1 = p1.reshape(e // 2, 128)
        p2 = p2.reshape(e // 2, 128)
        edge_fulls = [em["w1"][2 * c:], em["b1"][None, :], em["w2"],
                      em["b2"][None, :], em["ln_w"][None, :],
                      em["ln_b"][None, :]]
        if li == 0:
            en = _tc_mlp(
                _edge1_body, e, 2000,
                [p1, p2, ea16],
                [w1p, pe["b1"][None, :], pe["w2"], pe["b2"][None, :],
                 pe["ln_w"][None, :], pe["ln_b"][None, :]] + edge_fulls,
                1,
            )
        else:
            en = _tc_mlp(_edge_body, e, 2000, [p1, p2, en], edge_fulls, 1)
        agg2 = scatter_add(en, dstk3, zeros_nc)
        node_fulls = [nm["w1"][:c], nm["w1"][c:], nm["b1"][None, :], nm["w2"],
                      nm["b2"][None, :], nm["ln_w"][None, :],
                      nm["ln_b"][None, :]]
        if li + 1 < len(blocks):
            emn = blocks[li + 1]["edge_mlp"]
            grid = n // 2000
            x, y1, y2 = pl.pallas_call(
                _node_y_body,
                grid=(grid,),
                in_specs=[_row_spec(2000, 128)] * 3
                + [_full_spec(*a.shape)
                   for a in node_fulls + [emn["w1"][:c], emn["w1"][c:2 * c]]],
                out_specs=[_row_spec(2000, 128), _row_spec(2000, 64),
                           _row_spec(2000, 64)],
                out_shape=[
                    jax.ShapeDtypeStruct((n, 128), jnp.float32),
                    jax.ShapeDtypeStruct((n, 64), jnp.int32),
                    jax.ShapeDtypeStruct((n, 64), jnp.int32),
                ],
            )(x, agg2[0], agg2[1], *node_fulls, emn["w1"][:c],
              emn["w1"][c:2 * c])
        else:
            x = _tc_mlp(_node_body, n, 2000, [x, agg2[0], agg2[1]],
                        node_fulls, 1)
    return x
